# baseline (speedup 1.0000x reference)
import jax
import jax.numpy as jnp
import numpy as np
from jax.experimental import pallas as pl
from jax.experimental.pallas import tpu as pltpu

NUM_EXPERTS = 8
OUT = 1000
FEAT = 1728
IMG_FLAT = 3 * 384 * 384
CHUNK = 24576
N_CHUNKS = IMG_FLAT // CHUNK
FEAT_CHUNK = CHUNK // 256

_ph = np.arange(CHUNK) // 384 // 16
_pw = np.arange(CHUNK) % 384 // 16
_POOL_M = np.zeros((CHUNK, FEAT_CHUNK), np.float32)
_POOL_M[np.arange(CHUNK), _ph * 24 + _pw] = 1.0 / 256.0
_POOL_M = _POOL_M.astype(jnp.bfloat16)


def _pool_expert_kernel(x_ref, m_ref, we_ref, be_ref, z_ref, feat_ref):
    i = pl.program_id(0)

    @pl.when(i < N_CHUNKS)
    def _pool_phase():
        xb = x_ref[...]
        feat_ref[i] = jax.lax.dot_general(
            xb.astype(jnp.bfloat16), m_ref[...], (((1,), (0,)), ((), ())),
            preferred_element_type=jnp.float32)

    @pl.when(i >= N_CHUNKS)
    def _expert_phase():
        z = be_ref[0] * jnp.float32(1.0)
        for j in range(N_CHUNKS):
            z = z + jnp.dot(
                feat_ref[j], we_ref[0, j * FEAT_CHUNK:(j + 1) * FEAT_CHUNK],
                preferred_element_type=jnp.float32)
        z_ref[0] = z


def _gate_combine_kernel(x_ref, w_ref, z_ref, y_ref, logits_ref):
    i = pl.program_id(0)

    @pl.when(i < N_CHUNKS)
    def _gate_phase():
        part = jax.lax.dot_general(
            x_ref[...], w_ref[...], (((1,), (1,)), ((), ())),
            preferred_element_type=jnp.float32)

        @pl.when(i == 0)
        def _():
            logits_ref[...] = part

        @pl.when(i > 0)
        def _():
            logits_ref[...] += part

    @pl.when(i == N_CHUNKS)
    def _combine_phase():
        logits = logits_ref[...]
        B = logits.shape[0]
        m = jnp.max(logits, axis=1, keepdims=True)
        ex = jnp.exp(logits - m)
        v = ex / jnp.sum(ex, axis=1, keepdims=True)
        iota = jax.lax.broadcasted_iota(jnp.int32, (B, NUM_EXPERTS), 1)
        m1 = jnp.max(v, axis=1, keepdims=True)
        i1 = jnp.min(jnp.where(v == m1, iota, NUM_EXPERTS),
                     axis=1, keepdims=True)
        vm = jnp.where(iota == i1, jnp.float32(-jnp.inf), v)
        m2 = jnp.max(vm, axis=1, keepdims=True)
        i2 = jnp.min(jnp.where(vm == m2, iota, NUM_EXPERTS),
                     axis=1, keepdims=True)
        denom = m1 + m2 + 1e-6
        g = jnp.where(iota == i1, m1, jnp.where(iota == i2, m2, 0.0)) / denom

        y = g[:, 0:1] * z_ref[0]
        for e in range(1, NUM_EXPERTS):
            y = y + g[:, e:e + 1] * z_ref[e]
        y_ref[...] = y


def kernel(x, w_gate, w_noise, expert_W, expert_b):
    del w_noise
    B = x.shape[0]
    x_flat = x.reshape(B, IMG_FLAT)

    z = pl.pallas_call(
        _pool_expert_kernel,
        grid=(N_CHUNKS + NUM_EXPERTS,),
        in_specs=[
            pl.BlockSpec((B, CHUNK),
                         lambda i: (0, jnp.minimum(i, N_CHUNKS - 1))),
            pl.BlockSpec((CHUNK, FEAT_CHUNK), lambda i: (0, 0)),
            pl.BlockSpec((1, FEAT, OUT),
                         lambda i: (jnp.maximum(i - N_CHUNKS, 0), 0, 0)),
            pl.BlockSpec((1, 1, OUT),
                         lambda i: (jnp.maximum(i - N_CHUNKS, 0), 0, 0)),
        ],
        out_specs=pl.BlockSpec((1, B, OUT),
                               lambda i: (jnp.maximum(i - N_CHUNKS, 0), 0, 0)),
        out_shape=jax.ShapeDtypeStruct((NUM_EXPERTS, B, OUT), jnp.float32),
        scratch_shapes=[
            pltpu.VMEM((N_CHUNKS, B, FEAT_CHUNK), jnp.float32),
        ],
    )(x_flat, jnp.asarray(_POOL_M), expert_W,
      expert_b.reshape(NUM_EXPERTS, 1, OUT))

    y = pl.pallas_call(
        _gate_combine_kernel,
        grid=(N_CHUNKS + 1,),
        in_specs=[
            pl.BlockSpec((B, CHUNK),
                         lambda i: (0, jnp.minimum(i, N_CHUNKS - 1))),
            pl.BlockSpec((NUM_EXPERTS, CHUNK),
                         lambda i: (0, jnp.minimum(i, N_CHUNKS - 1))),
            pl.BlockSpec((NUM_EXPERTS, B, OUT), lambda i: (0, 0, 0)),
        ],
        out_specs=pl.BlockSpec((B, OUT), lambda i: (0, 0)),
        out_shape=jax.ShapeDtypeStruct((B, OUT), jnp.float32),
        scratch_shapes=[
            pltpu.VMEM((B, NUM_EXPERTS), jnp.float32),
        ],
    )(x_flat, w_gate.T, z)
    return y


# speedup vs baseline: 1.1273x; 1.1273x over previous
import jax
import jax.numpy as jnp
import numpy as np
from jax.experimental import pallas as pl
from jax.experimental.pallas import tpu as pltpu

NUM_EXPERTS = 8
OUT = 1000
FEAT = 1728
IMG_FLAT = 3 * 384 * 384
CHUNK = 24576
N_CHUNKS = IMG_FLAT // CHUNK
FEAT_CHUNK = CHUNK // 256

_ph = np.arange(CHUNK) // 384 // 16
_pw = np.arange(CHUNK) % 384 // 16
_POOL_M = np.zeros((CHUNK, FEAT_CHUNK), np.float32)
_POOL_M[np.arange(CHUNK), _ph * 24 + _pw] = 1.0 / 256.0
_POOL_M = _POOL_M.astype(jnp.bfloat16)


def _moe_kernel(x_ref, w_ref, m_ref, we_ref, b_ref, y_ref,
                logits_ref, feat_ref, z_ref):
    i = pl.program_id(0)

    @pl.when(i < N_CHUNKS)
    def _gate_pool():
        xb = x_ref[...]
        part = jax.lax.dot_general(
            xb, w_ref[...], (((1,), (1,)), ((), ())),
            preferred_element_type=jnp.float32)
        pooled = jax.lax.dot_general(
            xb.astype(jnp.bfloat16), m_ref[...], (((1,), (0,)), ((), ())),
            preferred_element_type=jnp.float32)
        feat_ref[i % 2] = pooled

        @pl.when(i == 0)
        def _():
            logits_ref[...] = part

        @pl.when(i > 0)
        def _():
            logits_ref[...] += part

    @pl.when(jnp.logical_and(i >= 1, i <= N_CHUNKS))
    def _expert_slab():
        f = feat_ref[(i - 1) % 2]
        for e in range(NUM_EXPERTS):
            part_z = jnp.dot(f, we_ref[e, 0],
                             preferred_element_type=jnp.float32)

            @pl.when(i == 1)
            def _():
                z_ref[e] = part_z

            @pl.when(i > 1)
            def _():
                z_ref[e] += part_z

    @pl.when(i == N_CHUNKS + 1)
    def _combine():
        logits = logits_ref[...]
        B = logits.shape[0]
        m = jnp.max(logits, axis=1, keepdims=True)
        ex = jnp.exp(logits - m)
        v = ex / jnp.sum(ex, axis=1, keepdims=True)
        iota = jax.lax.broadcasted_iota(jnp.int32, (B, NUM_EXPERTS), 1)
        m1 = jnp.max(v, axis=1, keepdims=True)
        i1 = jnp.min(jnp.where(v == m1, iota, NUM_EXPERTS),
                     axis=1, keepdims=True)
        vm = jnp.where(iota == i1, jnp.float32(-jnp.inf), v)
        m2 = jnp.max(vm, axis=1, keepdims=True)
        i2 = jnp.min(jnp.where(vm == m2, iota, NUM_EXPERTS),
                     axis=1, keepdims=True)
        denom = m1 + m2 + 1e-6
        g = jnp.where(iota == i1, m1,
                      jnp.where(iota == i2, m2, 0.0)) / denom

        y = g[:, 0:1] * (z_ref[0] + b_ref[0:1])
        for e in range(1, NUM_EXPERTS):
            y = y + g[:, e:e + 1] * (z_ref[e] + b_ref[e:e + 1])
        y_ref[...] = y


def kernel(x, w_gate, w_noise, expert_W, expert_b):
    del w_noise
    B = x.shape[0]
    x_flat = x.reshape(B, IMG_FLAT)
    we = expert_W.reshape(NUM_EXPERTS, N_CHUNKS, FEAT_CHUNK, OUT)

    y = pl.pallas_call(
        _moe_kernel,
        grid=(N_CHUNKS + 2,),
        in_specs=[
            pl.BlockSpec((B, CHUNK),
                         lambda i: (0, jnp.minimum(i, N_CHUNKS - 1))),
            pl.BlockSpec((NUM_EXPERTS, CHUNK),
                         lambda i: (0, jnp.minimum(i, N_CHUNKS - 1))),
            pl.BlockSpec((CHUNK, FEAT_CHUNK), lambda i: (0, 0)),
            pl.BlockSpec(
                (NUM_EXPERTS, 1, FEAT_CHUNK, OUT),
                lambda i: (0, jnp.clip(i - 1, 0, N_CHUNKS - 1), 0, 0)),
            pl.BlockSpec((NUM_EXPERTS, OUT), lambda i: (0, 0)),
        ],
        out_specs=pl.BlockSpec((B, OUT), lambda i: (0, 0)),
        out_shape=jax.ShapeDtypeStruct((B, OUT), jnp.float32),
        scratch_shapes=[
            pltpu.VMEM((B, NUM_EXPERTS), jnp.float32),
            pltpu.VMEM((2, B, FEAT_CHUNK), jnp.float32),
            pltpu.VMEM((NUM_EXPERTS, B, OUT), jnp.float32),
        ],
    )(x_flat, w_gate.T, jnp.asarray(_POOL_M), we, expert_b)
    return y


# speedup vs baseline: 1.1736x; 1.0411x over previous
import jax
import jax.numpy as jnp
import numpy as np
from jax.experimental import pallas as pl
from jax.experimental.pallas import tpu as pltpu

NUM_EXPERTS = 8
OUT = 1000
FEAT = 1728
IMG_FLAT = 3 * 384 * 384
CHUNK = 24576
N_CHUNKS = IMG_FLAT // CHUNK
FEAT_CHUNK = CHUNK // 256

_ph = np.arange(CHUNK) // 384 // 16
_pw = np.arange(CHUNK) % 384 // 16
_POOL_M = np.zeros((CHUNK, FEAT_CHUNK), np.float32)
_POOL_M[np.arange(CHUNK), _ph * 24 + _pw] = 1.0 / 256.0
_POOL_M = _POOL_M.astype(jnp.bfloat16)


def _moe_kernel(x_ref, w_ref, m_ref, we_ref, be_ref, y_ref,
                logits_ref, feat_ref):
    i = pl.program_id(0)

    @pl.when(i < N_CHUNKS)
    def _gate_pool_phase():
        xb = x_ref[...]
        part = jax.lax.dot_general(
            xb, w_ref[...], (((1,), (1,)), ((), ())),
            preferred_element_type=jnp.float32)
        pooled = jax.lax.dot_general(
            xb.astype(jnp.bfloat16), m_ref[...], (((1,), (0,)), ((), ())),
            preferred_element_type=jnp.float32)
        feat_ref[i] = pooled

        @pl.when(i == 0)
        def _():
            logits_ref[...] = part

        @pl.when(i > 0)
        def _():
            logits_ref[...] += part

    @pl.when(i >= N_CHUNKS)
    def _expert_phase():
        e = i - N_CHUNKS
        logits = logits_ref[...]
        B = logits.shape[0]
        m = jnp.max(logits, axis=1, keepdims=True)
        ex = jnp.exp(logits - m)
        v = ex / jnp.sum(ex, axis=1, keepdims=True)
        iota = jax.lax.broadcasted_iota(jnp.int32, (B, NUM_EXPERTS), 1)
        m1 = jnp.max(v, axis=1, keepdims=True)
        i1 = jnp.min(jnp.where(v == m1, iota, NUM_EXPERTS),
                     axis=1, keepdims=True)
        vm = jnp.where(iota == i1, jnp.float32(-jnp.inf), v)
        m2 = jnp.max(vm, axis=1, keepdims=True)
        i2 = jnp.min(jnp.where(vm == m2, iota, NUM_EXPERTS),
                     axis=1, keepdims=True)
        denom = m1 + m2 + 1e-6
        g = jnp.where(i1 == e, m1, jnp.where(i2 == e, m2, 0.0)) / denom

        z = be_ref[0] * jnp.float32(1.0)
        for j in range(N_CHUNKS):
            z = z + jnp.dot(
                feat_ref[j], we_ref[0, j * FEAT_CHUNK:(j + 1) * FEAT_CHUNK],
                preferred_element_type=jnp.float32)
        contrib = g * z

        @pl.when(e == 0)
        def _():
            y_ref[...] = contrib

        @pl.when(e > 0)
        def _():
            y_ref[...] += contrib


def kernel(x, w_gate, w_noise, expert_W, expert_b):
    del w_noise
    B = x.shape[0]
    x_flat = x.reshape(B, IMG_FLAT)
    n_steps = N_CHUNKS + NUM_EXPERTS

    y = pl.pallas_call(
        _moe_kernel,
        grid=(n_steps,),
        in_specs=[
            pl.BlockSpec((B, CHUNK),
                         lambda i: (0, jnp.minimum(i, N_CHUNKS - 1))),
            pl.BlockSpec((NUM_EXPERTS, CHUNK),
                         lambda i: (0, jnp.minimum(i, N_CHUNKS - 1))),
            pl.BlockSpec((CHUNK, FEAT_CHUNK), lambda i: (0, 0)),
            pl.BlockSpec((1, FEAT, OUT),
                         lambda i: (jnp.maximum(i - N_CHUNKS, 0), 0, 0)),
            pl.BlockSpec((1, 1, OUT),
                         lambda i: (jnp.maximum(i - N_CHUNKS, 0), 0, 0)),
        ],
        out_specs=pl.BlockSpec((B, OUT), lambda i: (0, 0)),
        out_shape=jax.ShapeDtypeStruct((B, OUT), jnp.float32),
        scratch_shapes=[
            pltpu.VMEM((B, NUM_EXPERTS), jnp.float32),
            pltpu.VMEM((N_CHUNKS, B, FEAT_CHUNK), jnp.float32),
        ],
    )(x_flat, w_gate.T, jnp.asarray(_POOL_M), expert_W,
      expert_b.reshape(NUM_EXPERTS, 1, OUT))
    return y
